# baseline (speedup 1.0000x reference)
import jax
import jax.numpy as jnp
from jax import lax
from jax.experimental import pallas as pl
from jax.experimental.pallas import tpu as pltpu
from jax.experimental.pallas import tpu_sc as plsc

B, S, V = 256, 100, 128
NW = 32
IPW = B // NW
CH = 10
NCH = S // CH
NG = V // 16
LO0 = S - CH


def _maxtree(vals):
    vals = list(vals)
    while len(vals) > 1:
        vals = [
            jnp.maximum(vals[k], vals[k + 1]) for k in range(0, len(vals) - 1, 2)
        ] + ([vals[-1]] if len(vals) % 2 else [])
    return vals[0]


def _sc_body(x_hbm, out_hbm, bufc, bufx, bufy, outa, m_ref, sems, sem2):
    wid = lax.axis_index("s") * 2 + lax.axis_index("c")
    b0 = wid * IPW
    lane = lax.iota(jnp.int32, 16)
    one16 = jnp.ones((16,), jnp.float32)
    zero16f = jnp.zeros((16,), jnp.float32)

    def _copies(i, lo, sem):
        b = b0 + i
        return [
            pltpu.make_async_copy(
                x_hbm.at[pl.ds(lo, CH), ch, buf_i[0]], buf_i[1].at[i], sem
            )
            for ch, buf_i in ((0, (b, bufc)), (1, (b, bufx)), (2, (b, bufy)))
        ]

    def prefetch(i, _):
        for cpy in _copies(i, LO0, sems.at[i]):
            cpy.start()
        return 0

    lax.fori_loop(0, IPW, prefetch, 0)

    def per_item(i, _):
        i_splat = jnp.full((16,), i, jnp.int32)

        def cond(carry):
            c, cmin = carry
            return jnp.logical_or(
                c == 0, jnp.logical_and(c < NCH, cmin == 0)
            )

        def chunk(carry):
            c, _ = carry
            lo = S - CH * (c + 1)
            first = c == 0

            @pl.when(first)
            def _wait0():
                for cpy in _copies(i, LO0, sems.at[i]):
                    cpy.wait()

            @pl.when(jnp.logical_not(first))
            def _fetch_older():
                for cpy in _copies(i, lo, sem2):
                    cpy.start()
                for cpy in _copies(i, lo, sem2):
                    cpy.wait()

            def per_group(g, cmin_acc):
                gl = g * 16
                vlane = gl + lane
                cands = [
                    jnp.where(bufc[i, r, pl.ds(gl, 16)] > 0.0, r + 1, 0)
                    for r in range(CH)
                ]
                mrel = _maxtree(cands)
                mc = jnp.where(mrel > 0, mrel + lo, 0)
                mo = jnp.where(first, 0, m_ref[pl.ds(gl, 16)])
                newly = jnp.logical_and(mo == 0, mc > 0)
                rrow = jnp.where(newly, mrel - 1, 0)
                x = plsc.load_gather(bufx, [i_splat, rrow, vlane])
                y = plsc.load_gather(bufy, [i_splat, rrow, vlane])
                xo = jnp.where(first, zero16f, outa[1, i, pl.ds(gl, 16)])
                yo = jnp.where(first, zero16f, outa[2, i, pl.ds(gl, 16)])
                outa[1, i, pl.ds(gl, 16)] = jnp.where(newly, x, xo)
                outa[2, i, pl.ds(gl, 16)] = jnp.where(newly, y, yo)
                outa[0, i, pl.ds(gl, 16)] = one16
                mn = jnp.where(mo > 0, mo, mc)
                m_ref[pl.ds(gl, 16)] = mn
                return jnp.minimum(cmin_acc, mn)

            cminv = lax.fori_loop(
                0, NG, per_group, jnp.full((16,), 2**30, jnp.int32)
            )
            return c + 1, jnp.min(cminv)

        lax.while_loop(cond, chunk, (0, 0))
        return 0

    lax.fori_loop(0, IPW, per_item, 0)
    pltpu.sync_copy(outa, out_hbm.at[:, pl.ds(b0, IPW)])


def kernel(batch):
    xt = jnp.transpose(batch, (1, 3, 0, 2))
    mesh = plsc.VectorSubcoreMesh(core_axis_name="c", subcore_axis_name="s")
    k = pl.kernel(
        _sc_body,
        out_type=jax.ShapeDtypeStruct((3, B, V), jnp.float32),
        mesh=mesh,
        scratch_types=[
            pltpu.VMEM((IPW, CH, V), jnp.float32),
            pltpu.VMEM((IPW, CH, V), jnp.float32),
            pltpu.VMEM((IPW, CH, V), jnp.float32),
            pltpu.VMEM((3, IPW, V), jnp.float32),
            pltpu.VMEM((V,), jnp.int32),
            pltpu.SemaphoreType.DMA((IPW,)),
            pltpu.SemaphoreType.DMA,
        ],
        compiler_params=pltpu.CompilerParams(
            needs_layout_passes=False, use_tc_tiling_on_sc=False
        ),
    )
    out = k(xt)
    return jnp.transpose(out, (1, 2, 0))
